# Initial kernel scaffold; baseline (speedup 1.0000x reference)
import jax
import jax.numpy as jnp
from jax.experimental import pallas as pl


def kernel(x, conv_w):
    raise NotImplementedError("write your pallas kernel here")


# speedup vs baseline: 8.8289x; 8.8289x over previous
import jax
import jax.numpy as jnp
from jax import lax
from jax.experimental import pallas as pl
from jax.experimental.pallas import tpu as pltpu

N = 128 * 128
K0 = N // 2
R, C = 128, 128


def _tc_body(w_ref, x0, x1, x2, x3, x4, x5, x6, newx_ref, score_ref):
    xs = (x0, x1, x2, x3, x4, x5, x6)
    xb = [xs[d][...].astype(jnp.bfloat16).astype(jnp.float32) for d in range(7)]
    wb = [w_ref[d].astype(jnp.bfloat16).astype(jnp.float32) for d in range(7)]
    t = [xb[d] * wb[d] for d in range(7)]
    y = (((t[0] + t[1]) + (t[2] + t[3])) + (t[4] + t[5])) + t[6]
    score = jax.nn.sigmoid(y)
    key = lax.bitcast_convert_type(score, jnp.int32)

    T = jnp.int32(0)
    for b in range(29, -1, -1):
        cand = T | jnp.int32(1 << b)
        c = jnp.sum((key < cand).astype(jnp.int32))
        T = jnp.where(c < K0, cand, T)
    c_lt = jnp.sum((key < T).astype(jnp.int32))
    m = K0 - c_lt
    tie = key == T
    idx = (
        lax.broadcasted_iota(jnp.int32, (R, C), 0) * C
        + lax.broadcasted_iota(jnp.int32, (R, C), 1)
    )
    J = jnp.int32(0)
    for b in range(13, -1, -1):
        cand = J | jnp.int32(1 << b)
        c = jnp.sum((tie & (idx < cand)).astype(jnp.int32))
        J = jnp.where(c < m, cand, J)
    sel = (key < T) | (tie & (idx <= J))
    newx_ref[...] = jnp.where(sel, x3[...] * (score + 1.0), 0.0)
    score_ref[...] = score


def kernel(x, conv_w):
    xp = jnp.pad(x, (3, 3))
    xs = [xp[d : d + N].reshape(R, C) for d in range(7)]
    w = conv_w.reshape(7)
    newx, score = pl.pallas_call(
        _tc_body,
        out_shape=(
            jax.ShapeDtypeStruct((R, C), jnp.float32),
            jax.ShapeDtypeStruct((R, C), jnp.float32),
        ),
        in_specs=[pl.BlockSpec(memory_space=pltpu.SMEM)]
        + [pl.BlockSpec(memory_space=pltpu.VMEM)] * 7,
        out_specs=(
            pl.BlockSpec(memory_space=pltpu.VMEM),
            pl.BlockSpec(memory_space=pltpu.VMEM),
        ),
    )(w, *xs)
    return newx.reshape(N), score.reshape(N)
